# baseline (speedup 1.0000x reference)
import functools

import jax
import jax.numpy as jnp
from jax import lax
from jax.experimental import pallas as pl
from jax.experimental.pallas import tpu as pltpu
from jax.experimental.pallas import tpu_sc as plsc

D = 64
DP = 128
HIST = 50
HP = 56
BATCH = 16384
NC = 2
NS = 16
NW = NC * NS
IPW = BATCH // NW
CH = 4
NCHUNK = IPW // CH
NBUF = 2
NGRP = NCHUNK // NBUF


@functools.partial(
    pl.kernel,
    out_type=jax.ShapeDtypeStruct((BATCH, HIST, D), jnp.float32),
    mesh=plsc.VectorSubcoreMesh(core_axis_name="c", subcore_axis_name="s"),
    compiler_params=pltpu.CompilerParams(use_tc_tiling_on_sc=False),
    scratch_types=(
        [pltpu.VMEM((IPW, HP), jnp.int32)]
        + [pltpu.VMEM((CH, HP, DP), jnp.float32) for _ in range(NBUF)]
        + [pltpu.SemaphoreType.DMA for _ in range(2 * NBUF)]
    ),
)
def _emb_lookup(idx_hbm, table_hbm, out_hbm, idx_v, *bufs):
    rows = bufs[:NBUF]
    gsem = bufs[NBUF:2 * NBUF]
    wsem = bufs[2 * NBUF:]
    wid = lax.axis_index("s") * NC + lax.axis_index("c")
    ibase = wid * IPW
    pltpu.sync_copy(idx_hbm.at[wid], idx_v)

    def fire(c, b):
        for m in range(CH):
            pltpu.async_copy(
                table_hbm.at[idx_v.at[c * CH + m]],
                rows[b].at[m], gsem[b])

    def drain(c, b):
        for m in range(CH):
            pltpu.make_async_copy(
                table_hbm.at[idx_v.at[c * CH + m]],
                rows[b].at[m], gsem[b]).wait()

    def wcopy(c, b):
        src = rows[b].at[pl.ds(0, CH), pl.ds(0, HIST), pl.ds(0, D)]
        return src, out_hbm.at[pl.ds(ibase + c * CH, CH)]

    for b in range(NBUF):
        fire(b, b)

    def grp_body(grp, carry):
        for b in range(NBUF):
            c = grp * NBUF + b
            drain(c, b)
            src, dst = wcopy(c, b)
            pltpu.async_copy(src, dst, wsem[b])

            @pl.when(grp < NGRP - 1)
            def _():
                pltpu.make_async_copy(src, dst, wsem[b]).wait()
                fire(c + NBUF, b)

        return carry

    lax.fori_loop(0, NGRP, grp_body, 0)

    for b in range(NBUF):
        src, dst = wcopy(NCHUNK - NBUF + b, b)
        pltpu.make_async_copy(src, dst, wsem[b]).wait()


def kernel(input_ids, weight):
    idx = input_ids.astype(jnp.int32).reshape(NW, IPW, HIST)
    idx = jnp.pad(idx, ((0, 0), (0, 0), (0, HP - HIST)))
    table = jnp.pad(weight, ((0, 0), (0, DP - D)))
    return _emb_lookup(idx, table)


# speedup vs baseline: 1.5552x; 1.5552x over previous
import functools

import jax
import jax.numpy as jnp
from jax import lax
from jax.experimental import pallas as pl
from jax.experimental.pallas import tpu as pltpu
from jax.experimental.pallas import tpu_sc as plsc

D = 64
HIST = 50
HP = 56
BATCH = 16384
NC = 2
NS = 16
NW = NC * NS
IPW = BATCH // NW
CH = 8
NCHUNK = IPW // CH
NBUF = 2
NGRP = NCHUNK // NBUF


@functools.partial(
    pl.kernel,
    out_type=jax.ShapeDtypeStruct((BATCH, HIST, D), jnp.float32),
    mesh=plsc.VectorSubcoreMesh(core_axis_name="c", subcore_axis_name="s"),
    compiler_params=pltpu.CompilerParams(use_tc_tiling_on_sc=False),
    scratch_types=(
        [pltpu.VMEM((IPW, HP), jnp.int32)]
        + [pltpu.VMEM((CH, HP, D), jnp.float32) for _ in range(NBUF)]
        + [pltpu.SemaphoreType.DMA for _ in range(2 * NBUF)]
    ),
)
def _emb_lookup(idx_hbm, table_hbm, out_hbm, idx_v, *bufs):
    rows = bufs[:NBUF]
    gsem = bufs[NBUF:2 * NBUF]
    wsem = bufs[2 * NBUF:]
    wid = lax.axis_index("s") * NC + lax.axis_index("c")
    ibase = wid * IPW
    pltpu.sync_copy(idx_hbm.at[wid], idx_v)

    def fire(c, b):
        for m in range(CH):
            pltpu.async_copy(
                table_hbm.at[idx_v.at[c * CH + m]],
                rows[b].at[m], gsem[b])

    def drain(c, b):
        for m in range(CH):
            pltpu.make_async_copy(
                table_hbm.at[idx_v.at[c * CH + m]],
                rows[b].at[m], gsem[b]).wait()

    def wcopy(c, b):
        src = rows[b].at[pl.ds(0, CH), pl.ds(0, HIST), pl.ds(0, D)]
        return src, out_hbm.at[pl.ds(ibase + c * CH, CH)]

    for b in range(NBUF):
        fire(b, b)

    def grp_body(grp, carry):
        for b in range(NBUF):
            c = grp * NBUF + b
            drain(c, b)
            src, dst = wcopy(c, b)
            pltpu.async_copy(src, dst, wsem[b])

            @pl.when(grp < NGRP - 1)
            def _():
                pltpu.make_async_copy(src, dst, wsem[b]).wait()
                fire(c + NBUF, b)

        return carry

    lax.fori_loop(0, NGRP, grp_body, 0)

    for b in range(NBUF):
        src, dst = wcopy(NCHUNK - NBUF + b, b)
        pltpu.make_async_copy(src, dst, wsem[b]).wait()


def kernel(input_ids, weight):
    idx = input_ids.astype(jnp.int32).reshape(NW, IPW, HIST)
    idx = jnp.pad(idx, ((0, 0), (0, 0), (0, HP - HIST)))
    return _emb_lookup(idx, weight)


# speedup vs baseline: 4.2382x; 2.7252x over previous
import functools

import jax
import jax.numpy as jnp
from jax import lax
from jax.experimental import pallas as pl
from jax.experimental.pallas import tpu as pltpu
from jax.experimental.pallas import tpu_sc as plsc

D = 64
HIST = 50
BATCH = 16384
NC = 2
NS = 16
NW = NC * NS
IPW = BATCH // NW
CH = 8
L = CH * HIST
NCHUNK = IPW // CH
NBUF = 2
NGRP = NCHUNK // NBUF


@functools.partial(
    pl.kernel,
    out_type=jax.ShapeDtypeStruct((BATCH, HIST, D), jnp.float32),
    mesh=plsc.VectorSubcoreMesh(core_axis_name="c", subcore_axis_name="s"),
    compiler_params=pltpu.CompilerParams(use_tc_tiling_on_sc=False),
    scratch_types=(
        [pltpu.VMEM((NCHUNK, L), jnp.int32)]
        + [pltpu.VMEM((L, D), jnp.float32) for _ in range(NBUF)]
        + [pltpu.SemaphoreType.DMA for _ in range(2 * NBUF)]
    ),
)
def _emb_lookup(idx_hbm, table_hbm, out_hbm, idx_v, *bufs):
    rows = bufs[:NBUF]
    gsem = bufs[NBUF:2 * NBUF]
    wsem = bufs[2 * NBUF:]
    wid = lax.axis_index("s") * NC + lax.axis_index("c")
    ibase = wid * IPW
    pltpu.sync_copy(idx_hbm.at[wid], idx_v)

    def gdesc(c, b):
        return table_hbm.at[idx_v.at[c]], rows[b]

    def wdescs(c, b):
        return [(rows[b].at[pl.ds(m * HIST, HIST)],
                 out_hbm.at[ibase + c * CH + m]) for m in range(CH)]

    for b in range(NBUF):
        src, dst = gdesc(b, b)
        pltpu.async_copy(src, dst, gsem[b])

    def grp_body(grp, carry):
        for b in range(NBUF):
            c = grp * NBUF + b
            gs, gd = gdesc(c, b)
            pltpu.make_async_copy(gs, gd, gsem[b]).wait()
            wpairs = wdescs(c, b)
            for ws, wd in wpairs:
                pltpu.async_copy(ws, wd, wsem[b])

            @pl.when(grp < NGRP - 1)
            def _():
                for ws, wd in wpairs:
                    pltpu.make_async_copy(ws, wd, wsem[b]).wait()
                ns, nd = gdesc(c + NBUF, b)
                pltpu.async_copy(ns, nd, gsem[b])

        return carry

    lax.fori_loop(0, NGRP, grp_body, 0)

    for b in range(NBUF):
        for ws, wd in wdescs(NCHUNK - NBUF + b, b):
            pltpu.make_async_copy(ws, wd, wsem[b]).wait()


def kernel(input_ids, weight):
    idx = input_ids.astype(jnp.int32).reshape(NW, NCHUNK, L)
    return _emb_lookup(idx, weight)
